# Initial kernel scaffold; baseline (speedup 1.0000x reference)
import jax
import jax.numpy as jnp
from jax.experimental import pallas as pl


def kernel(opponent_action, weights):
    raise NotImplementedError("write your pallas kernel here")


# speedup vs baseline: 1.3646x; 1.3646x over previous
import functools

import jax
import jax.numpy as jnp
from jax import lax
from jax.experimental import pallas as pl
from jax.experimental.pallas import tpu as pltpu
from jax.experimental.pallas import tpu_sc as plsc

A = 8192
B = 4096

NC = 2
NS = 16
LANES = 16
NW = NC * NS
JW = A // NW
CH = 4
NCHUNK = JW // CH


def _sc_gather_body(opp_hbm, w_hbm, outt_hbm, idx_v, stage0, stage1,
                    frag0, frag1, sem_in0, sem_in1, sem_out0, sem_out1):
    wid = lax.axis_index("s") * NC + lax.axis_index("c")
    j0 = wid * JW

    pltpu.sync_copy(opp_hbm, idx_v)

    stages = (stage0, stage1)
    frags = (frag0, frag1)
    sems_in = (sem_in0, sem_in1)
    sems_out = (sem_out0, sem_out1)

    def start_in(c):
        buf = c & 1
        cp = pltpu.make_async_copy(
            w_hbm.at[pl.ds(j0 + c * CH, CH), :], stages[buf], sems_in[buf])
        cp.start()
        return cp

    in_cp = [None, None]
    out_cp = [None, None]
    in_cp[0] = start_in(0)

    for c in range(NCHUNK):
        buf = c & 1
        if c + 1 < NCHUNK:
            in_cp[1 - buf] = start_in(c + 1)
        in_cp[buf].wait()
        if out_cp[buf] is not None:
            out_cp[buf].wait()

        def gather_one(k, _, buf=buf):
            iv = idx_v[pl.ds(k * LANES, LANES)]
            for r in range(CH):
                rv = jnp.full((LANES,), r, jnp.int32)
                vals = plsc.load_gather(stages[buf], [rv, iv])
                frags[buf][r, pl.ds(k * LANES, LANES)] = vals
            return _

        lax.fori_loop(0, B // LANES, gather_one, None)

        cp = pltpu.make_async_copy(
            frags[buf], outt_hbm.at[pl.ds(j0 + c * CH, CH), :],
            sems_out[buf])
        cp.start()
        out_cp[buf] = cp

    out_cp[0].wait()
    out_cp[1].wait()


def _sc_gather(opp, weights):
    mesh = plsc.VectorSubcoreMesh(core_axis_name="c", subcore_axis_name="s")
    fn = pl.kernel(
        _sc_gather_body,
        out_type=jax.ShapeDtypeStruct((A, B), jnp.float32),
        mesh=mesh,
        compiler_params=pltpu.CompilerParams(needs_layout_passes=False),
        scratch_types=[
            pltpu.VMEM((B,), jnp.int32),
            pltpu.VMEM((CH, A), jnp.float32),
            pltpu.VMEM((CH, A), jnp.float32),
            pltpu.VMEM((CH, B), jnp.float32),
            pltpu.VMEM((CH, B), jnp.float32),
            pltpu.SemaphoreType.DMA,
            pltpu.SemaphoreType.DMA,
            pltpu.SemaphoreType.DMA,
            pltpu.SemaphoreType.DMA,
        ],
    )
    return fn(opp, weights)


BB = 512
JB = 512


def _tc_sample_body(outt_ref, g_ref, pol_ref, act_ref, max_sc, idx_sc):
    j = pl.program_id(1)
    nj = pl.num_programs(1)

    p = outt_ref[...].T
    pol_ref[...] = p
    s = jnp.log(p + jnp.float32(1e-9)) + g_ref[...]

    bmax = jnp.max(s, axis=1, keepdims=True)
    jidx = lax.broadcasted_iota(jnp.int32, (BB, JB), 1) + j * JB
    cand = jnp.min(jnp.where(s == bmax, jidx, jnp.int32(2**30)),
                   axis=1, keepdims=True)

    @pl.when(j == 0)
    def _():
        max_sc[...] = bmax
        idx_sc[...] = cand

    @pl.when(j > 0)
    def _():
        upd = bmax > max_sc[...]
        idx_sc[...] = jnp.where(upd, cand, idx_sc[...])
        max_sc[...] = jnp.where(upd, bmax, max_sc[...])

    @pl.when(j == nj - 1)
    def _():
        act_ref[...] = idx_sc[...]


def _tc_sample(outt, g):
    grid = (B // BB, A // JB)
    return pl.pallas_call(
        _tc_sample_body,
        grid=grid,
        in_specs=[
            pl.BlockSpec((JB, BB), lambda b, j: (j, b)),
            pl.BlockSpec((BB, JB), lambda b, j: (b, j)),
        ],
        out_specs=[
            pl.BlockSpec((BB, JB), lambda b, j: (b, j)),
            pl.BlockSpec((BB, 1), lambda b, j: (b, 0)),
        ],
        out_shape=[
            jax.ShapeDtypeStruct((B, A), jnp.float32),
            jax.ShapeDtypeStruct((B, 1), jnp.int32),
        ],
        scratch_shapes=[
            pltpu.VMEM((BB, 1), jnp.float32),
            pltpu.VMEM((BB, 1), jnp.int32),
        ],
    )(outt, g)


@jax.jit
def kernel(opponent_action, weights):
    opp = opponent_action.astype(jnp.int32)
    g = jax.random.gumbel(jax.random.key(42), (B, A), jnp.float32)
    outt = _sc_gather(opp, weights)
    policy_cols, actions = _tc_sample(outt, g)
    return (actions.reshape(B), policy_cols)
